# baseline (speedup 1.0000x reference)
import functools

import jax
import jax.numpy as jnp
from jax import lax
from jax.experimental import pallas as pl
from jax.experimental.pallas import tpu as pltpu

TN = 256
DEPTH = 3


def _blk_copy(lower_hbm, buf, sems, step, nblk, slot):
    row = jnp.where(step < nblk, step, step - nblk) * TN
    return pltpu.make_async_copy(
        lower_hbm.at[pl.ds(row, TN), :], buf.at[slot], sems.at[slot])


def _fused_kernel(B, HD, nblk, lower_hbm, x_ref, W10_ref, W11_ref, b1_ref,
                  W20_ref, W21_ref, b2_ref, We_ref, be_ref, Wo_ref, bo_ref,
                  out_ref, h_ref, m_ref, buf, sems):
    i = pl.program_id(0)
    nsteps = 2 * nblk
    slot = lax.rem(i, DEPTH)

    @pl.when(i == 0)
    def _prologue():
        for d in range(DEPTH):
            _blk_copy(lower_hbm, buf, sems, jnp.int32(d), nblk,
                      jnp.int32(d)).start()

    _blk_copy(lower_hbm, buf, sems, i, nblk, slot).wait()
    L = buf[slot]

    @pl.when(i < nblk)
    def _phase1():
        sT = lax.dot_general(x_ref[...], L, (((1,), (1,)), ((), ())),
                             preferred_element_type=jnp.float32)
        s = sT.T
        xr = x_ref[:, pl.ds(i * TN, TN)].T
        W10 = W10_ref[...]
        W11 = W11_ref[...]
        b1 = b1_ref[...]
        pieces = []
        for b in range(B):
            hb = xr[:, b:b + 1] * W10 + s[:, b:b + 1] * W11 + b1
            pieces.append(jnp.maximum(hb, 0.0))
        h_ref[pl.ds(i * TN, TN), :] = jnp.concatenate(pieces, axis=1)

    @pl.when(i >= nblk)
    def _phase2():
        j = i - nblk
        agg = jnp.dot(L, h_ref[...],
                      preferred_element_type=jnp.float32)
        Hi = h_ref[pl.ds(j * TN, TN), :]
        A = agg.reshape(TN * B, HD)
        Hf = Hi.reshape(TN * B, HD)
        G = (jnp.dot(Hf, W20_ref[...], preferred_element_type=jnp.float32)
             + jnp.dot(A, W21_ref[...], preferred_element_type=jnp.float32)
             + b2_ref[...])
        G = jnp.maximum(G, 0.0)
        Gm = jnp.max(G.reshape(TN, B * HD), axis=0, keepdims=True)

        @pl.when(j == 0)
        def _():
            m_ref[...] = Gm

        @pl.when(j > 0)
        def _():
            m_ref[...] = jnp.maximum(m_ref[...], Gm)

        @pl.when(j == nblk - 1)
        def _():
            mm = m_ref[...].reshape(B, HD)
            t = jnp.dot(mm, We_ref[...], preferred_element_type=jnp.float32)
            t = jnp.maximum(t + be_ref[...], 0.0)
            out_ref[...] = (jnp.dot(t, Wo_ref[...],
                                    preferred_element_type=jnp.float32)
                            + bo_ref[...])

    nxt = i + DEPTH

    @pl.when(nxt < nsteps)
    def _prefetch():
        _blk_copy(lower_hbm, buf, sems, nxt, nblk, slot).start()


def kernel(x, lower, _, W1_0, W1_1, b1, W2_0, W2_1, b2, We, be, Wo, bo):
    B, N, _d = x.shape
    HD = W1_0.shape[1]
    HFF = We.shape[1]
    NC = Wo.shape[1]
    nblk = N // TN

    x2d = x[:, :, 0]
    b1r = b1.reshape(1, HD)
    b2r = b2.reshape(1, HD)
    ber = be.reshape(1, HFF)
    bor = bo.reshape(1, NC)

    cidx = lambda i: (0, 0)
    out = pl.pallas_call(
        functools.partial(_fused_kernel, B, HD, nblk),
        grid=(2 * nblk,),
        in_specs=[
            pl.BlockSpec(memory_space=pl.ANY),
            pl.BlockSpec((B, N), cidx),
            pl.BlockSpec((1, HD), cidx),
            pl.BlockSpec((1, HD), cidx),
            pl.BlockSpec((1, HD), cidx),
            pl.BlockSpec((HD, HD), cidx),
            pl.BlockSpec((HD, HD), cidx),
            pl.BlockSpec((1, HD), cidx),
            pl.BlockSpec((HD, HFF), cidx),
            pl.BlockSpec((1, HFF), cidx),
            pl.BlockSpec((HFF, NC), cidx),
            pl.BlockSpec((1, NC), cidx),
        ],
        out_specs=pl.BlockSpec((B, NC), cidx),
        out_shape=jax.ShapeDtypeStruct((B, NC), jnp.float32),
        scratch_shapes=[
            pltpu.VMEM((N, B * HD), jnp.float32),
            pltpu.VMEM((1, B * HD), jnp.float32),
            pltpu.VMEM((DEPTH, TN, N), jnp.float32),
            pltpu.SemaphoreType.DMA((DEPTH,)),
        ],
    )(lower, x2d, W1_0, W1_1, b1r, W2_0, W2_1, b2r, We, ber, Wo, bor)

    return out


# speedup vs baseline: 1.0322x; 1.0322x over previous
import functools

import jax
import jax.numpy as jnp
from jax import lax
from jax.experimental import pallas as pl
from jax.experimental.pallas import tpu as pltpu

TN = 256


def _blk_copy(lower_hbm, buf, sems, blk):
    return pltpu.make_async_copy(
        lower_hbm.at[pl.ds(blk * TN, TN), :], buf.at[blk], sems.at[blk])


def _fused_kernel(B, HD, nblk, lower_hbm, x_ref, W10_ref, W11_ref, b1_ref,
                  W20_ref, W21_ref, b2_ref, We_ref, be_ref, Wo_ref, bo_ref,
                  out_ref, h_ref, m_ref, buf, sems):
    i = pl.program_id(0)

    @pl.when(i == 0)
    def _prologue():
        for d in range(nblk):
            _blk_copy(lower_hbm, buf, sems, jnp.int32(d)).start()

    @pl.when(i < nblk)
    def _phase1():
        _blk_copy(lower_hbm, buf, sems, i).wait()
        L = buf[i]
        sT = lax.dot_general(x_ref[...], L, (((1,), (1,)), ((), ())),
                             preferred_element_type=jnp.float32)
        s = sT.T
        xr = x_ref[:, pl.ds(i * TN, TN)].T
        W10 = W10_ref[...]
        W11 = W11_ref[...]
        b1 = b1_ref[...]
        pieces = []
        for b in range(B):
            hb = xr[:, b:b + 1] * W10 + s[:, b:b + 1] * W11 + b1
            pieces.append(jnp.maximum(hb, 0.0))
        h_ref[pl.ds(i * TN, TN), :] = jnp.concatenate(pieces, axis=1)

    @pl.when(i >= nblk)
    def _phase2():
        j = i - nblk
        L = buf[j]
        agg = jnp.dot(L, h_ref[...],
                      preferred_element_type=jnp.float32)
        Hi = h_ref[pl.ds(j * TN, TN), :]
        A = agg.reshape(TN * B, HD)
        Hf = Hi.reshape(TN * B, HD)
        G = (jnp.dot(Hf, W20_ref[...], preferred_element_type=jnp.float32)
             + jnp.dot(A, W21_ref[...], preferred_element_type=jnp.float32)
             + b2_ref[...])
        G = jnp.maximum(G, 0.0)
        Gm = jnp.max(G.reshape(TN, B * HD), axis=0, keepdims=True)

        @pl.when(j == 0)
        def _():
            m_ref[...] = Gm

        @pl.when(j > 0)
        def _():
            m_ref[...] = jnp.maximum(m_ref[...], Gm)

        @pl.when(j == nblk - 1)
        def _():
            mm = m_ref[...].reshape(B, HD)
            t = jnp.dot(mm, We_ref[...], preferred_element_type=jnp.float32)
            t = jnp.maximum(t + be_ref[...], 0.0)
            out_ref[...] = (jnp.dot(t, Wo_ref[...],
                                    preferred_element_type=jnp.float32)
                            + bo_ref[...])


def kernel(x, lower, _, W1_0, W1_1, b1, W2_0, W2_1, b2, We, be, Wo, bo):
    B, N, _d = x.shape
    HD = W1_0.shape[1]
    HFF = We.shape[1]
    NC = Wo.shape[1]
    nblk = N // TN

    x2d = x[:, :, 0]
    b1r = b1.reshape(1, HD)
    b2r = b2.reshape(1, HD)
    ber = be.reshape(1, HFF)
    bor = bo.reshape(1, NC)

    cidx = lambda i: (0, 0)
    out = pl.pallas_call(
        functools.partial(_fused_kernel, B, HD, nblk),
        grid=(2 * nblk,),
        in_specs=[
            pl.BlockSpec(memory_space=pl.ANY),
            pl.BlockSpec((B, N), cidx),
            pl.BlockSpec((1, HD), cidx),
            pl.BlockSpec((1, HD), cidx),
            pl.BlockSpec((1, HD), cidx),
            pl.BlockSpec((HD, HD), cidx),
            pl.BlockSpec((HD, HD), cidx),
            pl.BlockSpec((1, HD), cidx),
            pl.BlockSpec((HD, HFF), cidx),
            pl.BlockSpec((1, HFF), cidx),
            pl.BlockSpec((HFF, NC), cidx),
            pl.BlockSpec((1, NC), cidx),
        ],
        out_specs=pl.BlockSpec((B, NC), cidx),
        out_shape=jax.ShapeDtypeStruct((B, NC), jnp.float32),
        scratch_shapes=[
            pltpu.VMEM((N, B * HD), jnp.float32),
            pltpu.VMEM((1, B * HD), jnp.float32),
            pltpu.VMEM((N // TN, TN, N), jnp.float32),
            pltpu.SemaphoreType.DMA((N // TN,)),
        ],
    )(lower, x2d, W1_0, W1_1, b1r, W2_0, W2_1, b2r, We, ber, Wo, bor)

    return out


# speedup vs baseline: 1.1164x; 1.0815x over previous
import functools

import jax
import jax.numpy as jnp
from jax import lax
from jax.experimental import pallas as pl
from jax.experimental.pallas import tpu as pltpu

TN = 512


def _blk_copy(lower_hbm, buf, sems, blk):
    return pltpu.make_async_copy(
        lower_hbm.at[pl.ds(blk * TN, TN), :], buf.at[blk], sems.at[blk])


def _fused_kernel(B, HD, nblk, lower_hbm, x_ref, W10_ref, W11_ref, b1_ref,
                  W20_ref, W21_ref, b2_ref, We_ref, be_ref, Wo_ref, bo_ref,
                  out_ref, h_ref, m_ref, buf, sems):
    i = pl.program_id(0)

    @pl.when(i == 0)
    def _prologue():
        for d in range(nblk):
            _blk_copy(lower_hbm, buf, sems, jnp.int32(d)).start()

    @pl.when(i < nblk)
    def _phase1():
        _blk_copy(lower_hbm, buf, sems, i).wait()
        L = buf[i]
        sT = lax.dot_general(x_ref[...], L, (((1,), (1,)), ((), ())),
                             preferred_element_type=jnp.float32)
        s = sT.T
        xr = x_ref[:, pl.ds(i * TN, TN)].T
        W10 = W10_ref[...]
        W11 = W11_ref[...]
        b1 = b1_ref[...]
        pieces = []
        for b in range(B):
            hb = xr[:, b:b + 1] * W10 + s[:, b:b + 1] * W11 + b1
            pieces.append(jnp.maximum(hb, 0.0))
        h_ref[pl.ds(i * TN, TN), :] = jnp.concatenate(pieces, axis=1)

    @pl.when(i >= nblk)
    def _phase2():
        j = i - nblk
        L = buf[j]
        agg = jnp.dot(L, h_ref[...],
                      preferred_element_type=jnp.float32)
        Hi = h_ref[pl.ds(j * TN, TN), :]
        A = agg.reshape(TN * B, HD)
        Hf = Hi.reshape(TN * B, HD)
        G = (jnp.dot(Hf, W20_ref[...], preferred_element_type=jnp.float32)
             + jnp.dot(A, W21_ref[...], preferred_element_type=jnp.float32)
             + b2_ref[...])
        G = jnp.maximum(G, 0.0)
        Gm = jnp.max(G.reshape(TN, B * HD), axis=0, keepdims=True)

        @pl.when(j == 0)
        def _():
            m_ref[...] = Gm

        @pl.when(j > 0)
        def _():
            m_ref[...] = jnp.maximum(m_ref[...], Gm)

        @pl.when(j == nblk - 1)
        def _():
            mm = m_ref[...].reshape(B, HD)
            t = jnp.dot(mm, We_ref[...], preferred_element_type=jnp.float32)
            t = jnp.maximum(t + be_ref[...], 0.0)
            out_ref[...] = (jnp.dot(t, Wo_ref[...],
                                    preferred_element_type=jnp.float32)
                            + bo_ref[...])


def kernel(x, lower, _, W1_0, W1_1, b1, W2_0, W2_1, b2, We, be, Wo, bo):
    B, N, _d = x.shape
    HD = W1_0.shape[1]
    HFF = We.shape[1]
    NC = Wo.shape[1]
    nblk = N // TN

    x2d = x[:, :, 0]
    b1r = b1.reshape(1, HD)
    b2r = b2.reshape(1, HD)
    ber = be.reshape(1, HFF)
    bor = bo.reshape(1, NC)

    cidx = lambda i: (0, 0)
    out = pl.pallas_call(
        functools.partial(_fused_kernel, B, HD, nblk),
        grid=(2 * nblk,),
        in_specs=[
            pl.BlockSpec(memory_space=pl.ANY),
            pl.BlockSpec((B, N), cidx),
            pl.BlockSpec((1, HD), cidx),
            pl.BlockSpec((1, HD), cidx),
            pl.BlockSpec((1, HD), cidx),
            pl.BlockSpec((HD, HD), cidx),
            pl.BlockSpec((HD, HD), cidx),
            pl.BlockSpec((1, HD), cidx),
            pl.BlockSpec((HD, HFF), cidx),
            pl.BlockSpec((1, HFF), cidx),
            pl.BlockSpec((HFF, NC), cidx),
            pl.BlockSpec((1, NC), cidx),
        ],
        out_specs=pl.BlockSpec((B, NC), cidx),
        out_shape=jax.ShapeDtypeStruct((B, NC), jnp.float32),
        scratch_shapes=[
            pltpu.VMEM((N, B * HD), jnp.float32),
            pltpu.VMEM((1, B * HD), jnp.float32),
            pltpu.VMEM((N // TN, TN, N), jnp.float32),
            pltpu.SemaphoreType.DMA((N // TN,)),
        ],
    )(lower, x2d, W1_0, W1_1, b1r, W2_0, W2_1, b2r, We, ber, Wo, bor)

    return out


# speedup vs baseline: 1.1362x; 1.0178x over previous
import functools

import jax
import jax.numpy as jnp
from jax import lax
from jax.experimental import pallas as pl
from jax.experimental.pallas import tpu as pltpu

TN = 1024


def _blk_copy(lower_hbm, buf, sems, blk):
    return pltpu.make_async_copy(
        lower_hbm.at[pl.ds(blk * TN, TN), :], buf.at[blk], sems.at[blk])


def _fused_kernel(B, HD, nblk, lower_hbm, x_ref, W10_ref, W11_ref, b1_ref,
                  W20_ref, W21_ref, b2_ref, We_ref, be_ref, Wo_ref, bo_ref,
                  out_ref, h_ref, m_ref, buf, sems):
    i = pl.program_id(0)

    @pl.when(i == 0)
    def _prologue():
        for d in range(nblk):
            _blk_copy(lower_hbm, buf, sems, jnp.int32(d)).start()

    @pl.when(i < nblk)
    def _phase1():
        _blk_copy(lower_hbm, buf, sems, i).wait()
        L = buf[i]
        sT = lax.dot_general(x_ref[...], L, (((1,), (1,)), ((), ())),
                             preferred_element_type=jnp.float32)
        s = sT.T
        xr = x_ref[:, pl.ds(i * TN, TN)].T
        W10 = W10_ref[...]
        W11 = W11_ref[...]
        b1 = b1_ref[...]
        pieces = []
        for b in range(B):
            hb = xr[:, b:b + 1] * W10 + s[:, b:b + 1] * W11 + b1
            pieces.append(jnp.maximum(hb, 0.0))
        h_ref[pl.ds(i * TN, TN), :] = jnp.concatenate(pieces, axis=1)

    @pl.when(i >= nblk)
    def _phase2():
        j = i - nblk
        L = buf[j]
        agg = jnp.dot(L, h_ref[...],
                      preferred_element_type=jnp.float32)
        Hi = h_ref[pl.ds(j * TN, TN), :]
        A = agg.reshape(TN * B, HD)
        Hf = Hi.reshape(TN * B, HD)
        G = (jnp.dot(Hf, W20_ref[...], preferred_element_type=jnp.float32)
             + jnp.dot(A, W21_ref[...], preferred_element_type=jnp.float32)
             + b2_ref[...])
        G = jnp.maximum(G, 0.0)
        Gm = jnp.max(G.reshape(TN, B * HD), axis=0, keepdims=True)

        @pl.when(j == 0)
        def _():
            m_ref[...] = Gm

        @pl.when(j > 0)
        def _():
            m_ref[...] = jnp.maximum(m_ref[...], Gm)

        @pl.when(j == nblk - 1)
        def _():
            mm = m_ref[...].reshape(B, HD)
            t = jnp.dot(mm, We_ref[...], preferred_element_type=jnp.float32)
            t = jnp.maximum(t + be_ref[...], 0.0)
            out_ref[...] = (jnp.dot(t, Wo_ref[...],
                                    preferred_element_type=jnp.float32)
                            + bo_ref[...])


def kernel(x, lower, _, W1_0, W1_1, b1, W2_0, W2_1, b2, We, be, Wo, bo):
    B, N, _d = x.shape
    HD = W1_0.shape[1]
    HFF = We.shape[1]
    NC = Wo.shape[1]
    nblk = N // TN

    x2d = x[:, :, 0]
    b1r = b1.reshape(1, HD)
    b2r = b2.reshape(1, HD)
    ber = be.reshape(1, HFF)
    bor = bo.reshape(1, NC)

    cidx = lambda i: (0, 0)
    out = pl.pallas_call(
        functools.partial(_fused_kernel, B, HD, nblk),
        grid=(2 * nblk,),
        in_specs=[
            pl.BlockSpec(memory_space=pl.ANY),
            pl.BlockSpec((B, N), cidx),
            pl.BlockSpec((1, HD), cidx),
            pl.BlockSpec((1, HD), cidx),
            pl.BlockSpec((1, HD), cidx),
            pl.BlockSpec((HD, HD), cidx),
            pl.BlockSpec((HD, HD), cidx),
            pl.BlockSpec((1, HD), cidx),
            pl.BlockSpec((HD, HFF), cidx),
            pl.BlockSpec((1, HFF), cidx),
            pl.BlockSpec((HFF, NC), cidx),
            pl.BlockSpec((1, NC), cidx),
        ],
        out_specs=pl.BlockSpec((B, NC), cidx),
        out_shape=jax.ShapeDtypeStruct((B, NC), jnp.float32),
        scratch_shapes=[
            pltpu.VMEM((N, B * HD), jnp.float32),
            pltpu.VMEM((1, B * HD), jnp.float32),
            pltpu.VMEM((N // TN, TN, N), jnp.float32),
            pltpu.SemaphoreType.DMA((N // TN,)),
        ],
    )(lower, x2d, W1_0, W1_1, b1r, W2_0, W2_1, b2r, We, ber, Wo, bor)

    return out


# speedup vs baseline: 1.2657x; 1.1140x over previous
import functools

import jax
import jax.numpy as jnp
from jax import lax
from jax.experimental import pallas as pl
from jax.experimental.pallas import tpu as pltpu

TN = 1024


def _blk_copy(lower_hbm, buf, sems, blk):
    return pltpu.make_async_copy(
        lower_hbm.at[pl.ds(blk * TN, TN), :], buf.at[blk], sems.at[blk])


def _fused_kernel(B, HD, nblk, lower_hbm, x_ref, W10_ref, W11_ref, b1_ref,
                  W20_ref, W21_ref, b2_ref, We_ref, be_ref, Wo_ref, bo_ref,
                  out_ref, hw0_ref, hw1_ref, m_ref, buf, sems):
    i = pl.program_id(0)

    @pl.when(i == 0)
    def _prologue():
        for d in range(nblk):
            _blk_copy(lower_hbm, buf, sems, jnp.int32(d)).start()

    @pl.when(i < nblk)
    def _phase1():
        _blk_copy(lower_hbm, buf, sems, i).wait()
        L = buf[i]
        sT = lax.dot_general(x_ref[...], L, (((1,), (1,)), ((), ())),
                             preferred_element_type=jnp.float32)
        s = sT.T
        xr = x_ref[:, pl.ds(i * TN, TN)].T
        W10 = W10_ref[...]
        W11 = W11_ref[...]
        b1 = b1_ref[...]
        W20 = W20_ref[...]
        W21 = W21_ref[...]
        b2 = b2_ref[...]
        p0, p1 = [], []
        for b in range(B):
            hb = xr[:, b:b + 1] * W10 + s[:, b:b + 1] * W11 + b1
            hb = jnp.maximum(hb, 0.0)
            p0.append(jnp.dot(hb, W20,
                              preferred_element_type=jnp.float32) + b2)
            p1.append(jnp.dot(hb, W21,
                              preferred_element_type=jnp.float32))
        hw0_ref[pl.ds(i * TN, TN), :] = jnp.concatenate(p0, axis=1)
        hw1_ref[pl.ds(i * TN, TN), :] = jnp.concatenate(p1, axis=1)

    @pl.when(i >= nblk)
    def _phase2():
        j = i - nblk
        L = buf[j]
        agg = jnp.dot(L, hw1_ref[...],
                      preferred_element_type=jnp.float32)
        G = jnp.maximum(hw0_ref[pl.ds(j * TN, TN), :] + agg, 0.0)
        Gm = jnp.max(G, axis=0, keepdims=True)

        @pl.when(j == 0)
        def _():
            m_ref[...] = Gm

        @pl.when(j > 0)
        def _():
            m_ref[...] = jnp.maximum(m_ref[...], Gm)

        @pl.when(j == nblk - 1)
        def _():
            mm = m_ref[...].reshape(B, HD)
            t = jnp.dot(mm, We_ref[...], preferred_element_type=jnp.float32)
            t = jnp.maximum(t + be_ref[...], 0.0)
            out_ref[...] = (jnp.dot(t, Wo_ref[...],
                                    preferred_element_type=jnp.float32)
                            + bo_ref[...])


def kernel(x, lower, _, W1_0, W1_1, b1, W2_0, W2_1, b2, We, be, Wo, bo):
    B, N, _d = x.shape
    HD = W1_0.shape[1]
    HFF = We.shape[1]
    NC = Wo.shape[1]
    nblk = N // TN

    x2d = x[:, :, 0]
    b1r = b1.reshape(1, HD)
    b2r = b2.reshape(1, HD)
    ber = be.reshape(1, HFF)
    bor = bo.reshape(1, NC)

    cidx = lambda i: (0, 0)
    out = pl.pallas_call(
        functools.partial(_fused_kernel, B, HD, nblk),
        grid=(2 * nblk,),
        in_specs=[
            pl.BlockSpec(memory_space=pl.ANY),
            pl.BlockSpec((B, N), cidx),
            pl.BlockSpec((1, HD), cidx),
            pl.BlockSpec((1, HD), cidx),
            pl.BlockSpec((1, HD), cidx),
            pl.BlockSpec((HD, HD), cidx),
            pl.BlockSpec((HD, HD), cidx),
            pl.BlockSpec((1, HD), cidx),
            pl.BlockSpec((HD, HFF), cidx),
            pl.BlockSpec((1, HFF), cidx),
            pl.BlockSpec((HFF, NC), cidx),
            pl.BlockSpec((1, NC), cidx),
        ],
        out_specs=pl.BlockSpec((B, NC), cidx),
        out_shape=jax.ShapeDtypeStruct((B, NC), jnp.float32),
        scratch_shapes=[
            pltpu.VMEM((N, B * HD), jnp.float32),
            pltpu.VMEM((N, B * HD), jnp.float32),
            pltpu.VMEM((1, B * HD), jnp.float32),
            pltpu.VMEM((N // TN, TN, N), jnp.float32),
            pltpu.SemaphoreType.DMA((N // TN,)),
        ],
    )(lower, x2d, W1_0, W1_1, b1r, W2_0, W2_1, b2r, We, ber, Wo, bor)

    return out
